# baseline (speedup 1.0000x reference)
import functools

import jax
import jax.numpy as jnp
from jax import lax
from jax.experimental import pallas as pl
from jax.experimental.pallas import tpu as pltpu
from jax.experimental.pallas import tpu_sc as plsc

_N = 10000
_E = 320000
_D = 128
_NC = 2
_NS = 16
_NW = _NC * _NS
_EPT = 10240
_EP = _NW * _EPT
_HALF = 5120
_SINK = 16
_AR = _HALF + _SINK
_SEG = _EPT + 32
_K = 8
_NB = 4
_NI = 8

_mesh = plsc.VectorSubcoreMesh(core_axis_name="c", subcore_axis_name="s")


@functools.partial(
    pl.kernel,
    mesh=_mesh,
    compiler_params=pltpu.CompilerParams(needs_layout_passes=False),
    out_type=(
        jax.ShapeDtypeStruct((2 * _NW * _SEG,), jnp.int32),
        jax.ShapeDtypeStruct((2 * _NW * _SEG,), jnp.int32),
        jax.ShapeDtypeStruct((32 * _NW,), jnp.int32),
    ),
    scratch_types=[
        pltpu.VMEM((_EPT,), jnp.int32),
        pltpu.VMEM((_EPT,), jnp.int32),
        [pltpu.VMEM((_SEG,), jnp.int32) for _ in range(2)],
        [pltpu.VMEM((_SEG,), jnp.int32) for _ in range(2)],
        pltpu.VMEM((32,), jnp.int32),
    ],
)
def _sc_partition(src_hbm, dst_hbm, sp_hbm, dp_hbm, cnt_hbm,
                  sbuf, dbuf, sc_, dc_, cbuf):
    cid = lax.axis_index("c")
    sid = lax.axis_index("s")
    wid = sid * _NC + cid

    pltpu.sync_copy(src_hbm.at[pl.ds(wid * _EPT, _EPT)], sbuf)
    pltpu.sync_copy(dst_hbm.at[pl.ds(wid * _EPT, _EPT)], dbuf)

    iota = lax.iota(jnp.int32, 16)

    def _body(j, carry):
        p0, p1 = carry
        s = sbuf[pl.ds(j * 16, 16)]
        d = dbuf[pl.ds(j * 16, 16)]
        m0 = d < _HALF
        cv = plsc.cumsum(m0.astype(jnp.int32))
        idx0 = (p0 - 1) + cv
        idx1 = (p1 + iota) - cv
        m1 = jnp.logical_not(m0)
        plsc.store_scatter(sc_[0], [idx0], s, mask=m0)
        plsc.store_scatter(dc_[0], [idx0], d, mask=m0)
        plsc.store_scatter(sc_[1], [idx1], s, mask=m1)
        plsc.store_scatter(dc_[1], [idx1], d - _HALF, mask=m1)
        n0 = jnp.max(cv)
        return p0 + n0, p1 + (16 - n0)

    p0, p1 = lax.fori_loop(0, _EPT // 16, _body, (0, 0))

    sinkd = _HALF + iota
    zsrc = jnp.zeros((16,), jnp.int32)
    for h, p in ((0, p0), (1, p1)):
        for o in (0, 16):
            plsc.store_scatter(dc_[h], [p + o + iota], sinkd)
            plsc.store_scatter(sc_[h], [p + o + iota], zsrc)
    p0c = ((p0 + 31) // 32) * 32
    p1c = ((p1 + 31) // 32) * 32
    cbuf[pl.ds(0, 16)] = jnp.full((16,), 0, jnp.int32) + p0c
    cbuf[pl.ds(16, 16)] = jnp.full((16,), 0, jnp.int32) + p1c

    pltpu.sync_copy(cbuf, cnt_hbm.at[pl.ds(wid * 32, 32)])
    for h in (0, 1):
        base = (wid * 2 + h) * _SEG
        pltpu.sync_copy(sc_[h], sp_hbm.at[pl.ds(base, _SEG)])
        pltpu.sync_copy(dc_[h], dp_hbm.at[pl.ds(base, _SEG)])


@functools.partial(
    pl.kernel,
    mesh=_mesh,
    compiler_params=pltpu.CompilerParams(needs_layout_passes=False),
    out_type=jax.ShapeDtypeStruct((_NC, _HALF, _D), jnp.float32),
    scratch_types=[
        [pltpu.VMEM((_K,), jnp.int32) for _ in range(_NI)],
        [pltpu.VMEM((_K,), jnp.int32) for _ in range(_NI)],
        pltpu.VMEM((_NB, _K, _D), jnp.float32),
        pltpu.VMEM_SHARED((_N, _D), jnp.float32),
        pltpu.VMEM_SHARED((_AR, _D), jnp.float32),
        [pltpu.SemaphoreType.DMA for _ in range(_NI)],
        [pltpu.SemaphoreType.DMA for _ in range(_NB)],
        [pltpu.SemaphoreType.DMA for _ in range(_NB)],
    ],
)
def _sc_aggregate(h_hbm, sp_hbm, dp_hbm, cnt_hbm, out_hbm,
                  src_i, dst_i, rows, h_sh, agg_sh, isems, rsems, ssems):
    cid = lax.axis_index("c")
    sid = lax.axis_index("s")

    cb0 = (4 * sid + cid) * _SEG
    cb1 = (4 * sid + 2 + cid) * _SEG

    pltpu.sync_copy(cnt_hbm.at[pl.ds((2 * sid) * 32 + cid * 16, 16)],
                    src_i[0].at[pl.ds(0, 16)])
    pltpu.sync_copy(cnt_hbm.at[pl.ds((2 * sid + 1) * 32 + cid * 16, 16)],
                    src_i[1].at[pl.ds(0, 16)])
    n0 = jnp.max(src_i[0][pl.ds(0, 16)])
    n1 = jnp.max(src_i[1][pl.ds(0, 16)])
    c0 = n0 // _K
    t = c0 + n1 // _K
    g_hi = (t + 2 + _NI - 1) // _NI

    def _off(j):
        return jnp.where(j < c0, cb0 + j * _K, cb1 + (j - c0) * _K)

    def _start_idx(j, slot):
        @pl.when(j < t)
        def _():
            o = _off(j)
            pltpu.async_copy(sp_hbm.at[pl.ds(o, _K)], src_i[slot], isems[slot])
            pltpu.async_copy(dp_hbm.at[pl.ds(o, _K)], dst_i[slot], isems[slot])

    def _wait_idx(j, slot):
        @pl.when(j < t)
        def _():
            pltpu.make_async_copy(sp_hbm.at[pl.ds(0, _K)], src_i[slot],
                                  isems[slot]).wait()
            pltpu.make_async_copy(dp_hbm.at[pl.ds(0, _K)], dst_i[slot],
                                  isems[slot]).wait()

    def _start_gather(j, slot, rslot):
        @pl.when(j < t)
        def _():
            pltpu.async_copy(h_sh.at[src_i[slot]], rows.at[rslot], rsems[rslot])

    def _wait_gather(j, rslot):
        @pl.when(j < t)
        def _():
            pltpu.make_async_copy(h_sh.at[src_i[0]], rows.at[rslot],
                                  rsems[rslot]).wait()

    def _start_scatter(j, slot, rslot):
        @pl.when(j < t)
        def _():
            pltpu.async_copy(rows.at[rslot], agg_sh.at[dst_i[slot]], ssems[rslot],
                             add=True)

    def _wait_scatter(j, rslot):
        @pl.when(jnp.logical_and(j >= 0, j < t))
        def _():
            pltpu.make_async_copy(rows.at[rslot], agg_sh.at[dst_i[0]],
                                  ssems[rslot]).wait()

    @pl.when(sid < 15)
    def _():
        pltpu.sync_copy(h_hbm.at[pl.ds(sid * 632, 632)],
                        h_sh.at[pl.ds(sid * 632, 632)])

    @pl.when(sid == 15)
    def _():
        pltpu.sync_copy(h_hbm.at[pl.ds(9480, 520)], h_sh.at[pl.ds(9480, 520)])

    def _zbody(i, carry):
        r = i // (_D // 16)
        c = (i % (_D // 16)) * 16
        rows[0, r, pl.ds(c, 16)] = jnp.zeros((16,), jnp.float32)
        return carry

    lax.fori_loop(0, _K * (_D // 16), _zbody, 0)
    zb = sid * (_AR // _NS)
    for q in range(_AR // _NS // _K):
        pltpu.sync_copy(rows.at[0], agg_sh.at[pl.ds(zb + q * _K, _K)])
    pltpu.sync_copy(rows.at[0, pl.ds(0, _AR // _NS % _K)],
                    agg_sh.at[pl.ds(zb + (_AR // _NS // _K) * _K,
                                    _AR // _NS % _K)])
    plsc.subcore_barrier()

    for j in range(6):
        _start_idx(j, j)
    for j in range(2):
        _wait_idx(j, j)
        _start_gather(j, j, j)

    def _body(g, carry):
        for b in range(_NI):
            j = g * _NI + b
            a = b % _NB

            _wait_gather(j, a)
            _start_scatter(j, b, a)
            _wait_scatter(j - 2, (b + 2) % _NB)
            _start_idx(j + 6, (b + 6) % _NI)
            _wait_idx(j + 2, (b + 2) % _NI)
            _start_gather(j + 2, (b + 2) % _NI, (b + 2) % _NB)
        return carry

    lax.fori_loop(0, g_hi, _body, 0)
    plsc.subcore_barrier()

    pltpu.sync_copy(agg_sh.at[pl.ds(sid * (_HALF // _NS), _HALF // _NS)],
                    out_hbm.at[cid, pl.ds(sid * (_HALF // _NS), _HALF // _NS)])


_RB = 1000


def _mlp_body(h_ref, p_ref, w1_ref, b1_ref, w2_ref, b2_ref, o_ref):
    z = h_ref[...] + p_ref[...]
    z = jnp.dot(z, w1_ref[...], preferred_element_type=jnp.float32) + b1_ref[...]
    z = jnp.maximum(z, 0.0)
    o_ref[...] = (jnp.dot(z, w2_ref[...], preferred_element_type=jnp.float32)
                  + b2_ref[...])


def _mlp(h, p, w1, b1, w2, b2):
    return pl.pallas_call(
        _mlp_body,
        grid=(_N // _RB,),
        in_specs=[
            pl.BlockSpec((_RB, _D), lambda i: (i, 0)),
            pl.BlockSpec((_RB, _D), lambda i: (i, 0)),
            pl.BlockSpec((_D, _D), lambda i: (0, 0)),
            pl.BlockSpec((1, _D), lambda i: (0, 0)),
            pl.BlockSpec((_D, _D), lambda i: (0, 0)),
            pl.BlockSpec((1, _D), lambda i: (0, 0)),
        ],
        out_specs=pl.BlockSpec((_RB, _D), lambda i: (i, 0)),
        out_shape=jax.ShapeDtypeStruct((_N, _D), jnp.float32),
    )(h, p, w1, b1.reshape(1, _D), w2, b2.reshape(1, _D))


def kernel(x, edge_index, W1_0, b1_0, W2_0, b2_0, W1_1, b1_1, W2_1, b2_1,
           W1_2, b1_2, W2_2, b2_2):
    pad = _EP - _E
    src = jnp.concatenate(
        [edge_index[0].astype(jnp.int32), jnp.zeros((pad,), jnp.int32)])
    dst = jnp.concatenate(
        [edge_index[1].astype(jnp.int32), jnp.full((pad,), _N, jnp.int32)])
    spf, dpf, cnt = _sc_partition(src, dst)
    params = [(W1_0, b1_0, W2_0, b2_0), (W1_1, b1_1, W2_1, b2_1),
              (W1_2, b1_2, W2_2, b2_2)]
    hs = [x]
    for (w1, b1, w2, b2) in params:
        p = _sc_aggregate(hs[-1], spf, dpf, cnt)
        p = p.reshape(_NC * _HALF, _D)[:_N]
        hs.append(_mlp(hs[-1], p, w1, b1, w2, b2))
    return jnp.concatenate(hs, axis=-1)


# speedup vs baseline: 1.1399x; 1.1399x over previous
import functools

import jax
import jax.numpy as jnp
from jax import lax
from jax.experimental import pallas as pl
from jax.experimental.pallas import tpu as pltpu
from jax.experimental.pallas import tpu_sc as plsc

_N = 10000
_E = 320000
_D = 128
_NC = 2
_NS = 16
_NW = _NC * _NS
_EPT = 10240
_EP = _NW * _EPT
_HALF = 5120
_SINK = 64
_AR = _HALF + _SINK
_SEG = _EPT + 32
_K = 32
_NB = 2
_NI = 4

_mesh = plsc.VectorSubcoreMesh(core_axis_name="c", subcore_axis_name="s")


@functools.partial(
    pl.kernel,
    mesh=_mesh,
    compiler_params=pltpu.CompilerParams(needs_layout_passes=False),
    out_type=(
        jax.ShapeDtypeStruct((2 * _NW * _SEG,), jnp.int32),
        jax.ShapeDtypeStruct((2 * _NW * _SEG,), jnp.int32),
        jax.ShapeDtypeStruct((32 * _NW,), jnp.int32),
    ),
    scratch_types=[
        pltpu.VMEM((_EPT,), jnp.int32),
        pltpu.VMEM((_EPT,), jnp.int32),
        [pltpu.VMEM((_SEG,), jnp.int32) for _ in range(2)],
        [pltpu.VMEM((_SEG,), jnp.int32) for _ in range(2)],
        pltpu.VMEM((32,), jnp.int32),
    ],
)
def _sc_partition(src_hbm, dst_hbm, sp_hbm, dp_hbm, cnt_hbm,
                  sbuf, dbuf, sc_, dc_, cbuf):
    cid = lax.axis_index("c")
    sid = lax.axis_index("s")
    wid = sid * _NC + cid

    pltpu.sync_copy(src_hbm.at[pl.ds(wid * _EPT, _EPT)], sbuf)
    pltpu.sync_copy(dst_hbm.at[pl.ds(wid * _EPT, _EPT)], dbuf)

    iota = lax.iota(jnp.int32, 16)

    def _body(j, carry):
        p0, p1 = carry
        s = sbuf[pl.ds(j * 16, 16)]
        d = dbuf[pl.ds(j * 16, 16)]
        m0 = d < _HALF
        cv = plsc.cumsum(m0.astype(jnp.int32))
        idx0 = (p0 - 1) + cv
        idx1 = (p1 + iota) - cv
        m1 = jnp.logical_not(m0)
        plsc.store_scatter(sc_[0], [idx0], s, mask=m0)
        plsc.store_scatter(dc_[0], [idx0], d, mask=m0)
        plsc.store_scatter(sc_[1], [idx1], s, mask=m1)
        plsc.store_scatter(dc_[1], [idx1], d - _HALF, mask=m1)
        n0 = jnp.max(cv)
        return p0 + n0, p1 + (16 - n0)

    p0, p1 = lax.fori_loop(0, _EPT // 16, _body, (0, 0))

    sinkd = _HALF + iota
    zsrc = jnp.zeros((16,), jnp.int32)
    for h, p in ((0, p0), (1, p1)):
        for o in (0, 16):
            plsc.store_scatter(dc_[h], [p + o + iota], sinkd)
            plsc.store_scatter(sc_[h], [p + o + iota], zsrc)
    p0c = ((p0 + 31) // 32) * 32
    p1c = ((p1 + 31) // 32) * 32
    cbuf[pl.ds(0, 16)] = jnp.full((16,), 0, jnp.int32) + p0c
    cbuf[pl.ds(16, 16)] = jnp.full((16,), 0, jnp.int32) + p1c

    pltpu.sync_copy(cbuf, cnt_hbm.at[pl.ds(wid * 32, 32)])
    for h in (0, 1):
        base = (wid * 2 + h) * _SEG
        pltpu.sync_copy(sc_[h], sp_hbm.at[pl.ds(base, _SEG)])
        pltpu.sync_copy(dc_[h], dp_hbm.at[pl.ds(base, _SEG)])


@functools.partial(
    pl.kernel,
    mesh=_mesh,
    compiler_params=pltpu.CompilerParams(needs_layout_passes=False),
    out_type=jax.ShapeDtypeStruct((_NC, _HALF, _D), jnp.float32),
    scratch_types=[
        [pltpu.VMEM((_K,), jnp.int32) for _ in range(_NI)],
        [pltpu.VMEM((_K,), jnp.int32) for _ in range(_NI)],
        pltpu.VMEM((_NB, _K, _D), jnp.float32),
        pltpu.VMEM_SHARED((_N, _D), jnp.float32),
        pltpu.VMEM_SHARED((_AR, _D), jnp.float32),
        [pltpu.SemaphoreType.DMA for _ in range(_NI)],
        [pltpu.SemaphoreType.DMA for _ in range(_NB)],
    ],
)
def _sc_aggregate(h_hbm, sp_hbm, dp_hbm, cnt_hbm, out_hbm,
                  src_i, dst_i, rows, h_sh, agg_sh, isems, rsems):
    cid = lax.axis_index("c")
    sid = lax.axis_index("s")

    cb0 = (4 * sid + cid) * _SEG
    cb1 = (4 * sid + 2 + cid) * _SEG

    pltpu.sync_copy(cnt_hbm.at[pl.ds((2 * sid) * 32 + cid * 16, 16)],
                    src_i[0].at[pl.ds(0, 16)])
    pltpu.sync_copy(cnt_hbm.at[pl.ds((2 * sid + 1) * 32 + cid * 16, 16)],
                    src_i[1].at[pl.ds(0, 16)])
    n0 = jnp.max(src_i[0][pl.ds(0, 16)])
    n1 = jnp.max(src_i[1][pl.ds(0, 16)])
    c0 = n0 // _K
    t = c0 + n1 // _K
    g_hi = (t + _NI - 1) // _NI

    def _off(j):
        return jnp.where(j < c0, cb0 + j * _K, cb1 + (j - c0) * _K)

    def _start_idx(j, slot):
        @pl.when(j < t)
        def _():
            o = _off(j)
            pltpu.async_copy(sp_hbm.at[pl.ds(o, _K)], src_i[slot], isems[slot])
            pltpu.async_copy(dp_hbm.at[pl.ds(o, _K)], dst_i[slot], isems[slot])

    def _wait_idx(j, slot):
        @pl.when(j < t)
        def _():
            pltpu.make_async_copy(sp_hbm.at[pl.ds(0, _K)], src_i[slot],
                                  isems[slot]).wait()
            pltpu.make_async_copy(dp_hbm.at[pl.ds(0, _K)], dst_i[slot],
                                  isems[slot]).wait()

    def _start_gather(j, slot, rslot):
        @pl.when(j < t)
        def _():
            pltpu.async_copy(h_sh.at[src_i[slot]], rows.at[rslot], rsems[rslot])

    def _wait_gather(j, rslot):
        @pl.when(j < t)
        def _():
            pltpu.make_async_copy(h_sh.at[src_i[0]], rows.at[rslot],
                                  rsems[rslot]).wait()

    @pl.when(sid < 15)
    def _():
        pltpu.sync_copy(h_hbm.at[pl.ds(sid * 632, 632)],
                        h_sh.at[pl.ds(sid * 632, 632)])

    @pl.when(sid == 15)
    def _():
        pltpu.sync_copy(h_hbm.at[pl.ds(9480, 520)], h_sh.at[pl.ds(9480, 520)])

    def _zbody(i, carry):
        r = i // (_D // 16)
        c = (i % (_D // 16)) * 16
        rows[0, r, pl.ds(c, 16)] = jnp.zeros((16,), jnp.float32)
        return carry

    lax.fori_loop(0, _K * (_D // 16), _zbody, 0)
    zb = sid * (_AR // _NS)
    for q in range(_AR // _NS // _K):
        pltpu.sync_copy(rows.at[0], agg_sh.at[pl.ds(zb + q * _K, _K)])
    pltpu.sync_copy(rows.at[0, pl.ds(0, _AR // _NS % _K)],
                    agg_sh.at[pl.ds(zb + (_AR // _NS // _K) * _K,
                                    _AR // _NS % _K)])
    plsc.subcore_barrier()

    for j in range(_NI):
        _start_idx(j, j)
    for j in range(_NB):
        _wait_idx(j, j)
        _start_gather(j, j, j)

    def _body(g, carry):
        for b in range(_NI):
            j = g * _NI + b
            rslot = b % _NB

            _wait_gather(j, rslot)

            @pl.when(j < t)
            def _():
                pltpu.sync_copy(rows.at[rslot], agg_sh.at[dst_i[b]], add=True)

            _start_idx(j + _NI, b)
            _wait_idx(j + _NB, (b + _NB) % _NI)
            _start_gather(j + _NB, (b + _NB) % _NI, rslot)
        return carry

    lax.fori_loop(0, g_hi, _body, 0)
    plsc.subcore_barrier()

    pltpu.sync_copy(agg_sh.at[pl.ds(sid * (_HALF // _NS), _HALF // _NS)],
                    out_hbm.at[cid, pl.ds(sid * (_HALF // _NS), _HALF // _NS)])


_NP = _NC * _HALF
_RB = 1024


def _mlp_body(h_ref, p_ref, w1_ref, b1_ref, w2_ref, b2_ref, o_ref):
    z = h_ref[...] + p_ref[0]
    z = jnp.dot(z, w1_ref[...], preferred_element_type=jnp.float32) + b1_ref[...]
    z = jnp.maximum(z, 0.0)
    o_ref[...] = (jnp.dot(z, w2_ref[...], preferred_element_type=jnp.float32)
                  + b2_ref[...])


def _mlp(h, p, w1, b1, w2, b2):
    return pl.pallas_call(
        _mlp_body,
        grid=(_NP // _RB,),
        in_specs=[
            pl.BlockSpec((_RB, _D), lambda i: (i, 0)),
            pl.BlockSpec((1, _RB, _D), lambda i: (i // 5, i % 5, 0)),
            pl.BlockSpec((_D, _D), lambda i: (0, 0)),
            pl.BlockSpec((1, _D), lambda i: (0, 0)),
            pl.BlockSpec((_D, _D), lambda i: (0, 0)),
            pl.BlockSpec((1, _D), lambda i: (0, 0)),
        ],
        out_specs=pl.BlockSpec((_RB, _D), lambda i: (i, 0)),
        out_shape=jax.ShapeDtypeStruct((_NP, _D), jnp.float32),
    )(h, p, w1, b1.reshape(1, _D), w2, b2.reshape(1, _D))


def kernel(x, edge_index, W1_0, b1_0, W2_0, b2_0, W1_1, b1_1, W2_1, b2_1,
           W1_2, b1_2, W2_2, b2_2):
    pad = _EP - _E
    src = jnp.concatenate(
        [edge_index[0].astype(jnp.int32), jnp.zeros((pad,), jnp.int32)])
    dst = jnp.concatenate(
        [edge_index[1].astype(jnp.int32), jnp.full((pad,), _N, jnp.int32)])
    spf, dpf, cnt = _sc_partition(src, dst)
    params = [(W1_0, b1_0, W2_0, b2_0), (W1_1, b1_1, W2_1, b2_1),
              (W1_2, b1_2, W2_2, b2_2)]
    x_p = jnp.concatenate([x, jnp.zeros((_NP - _N, _D), jnp.float32)])
    hs = [x_p]
    for (w1, b1, w2, b2) in params:
        p = _sc_aggregate(hs[-1], spf, dpf, cnt)
        hs.append(_mlp(hs[-1], p, w1, b1, w2, b2))
    return jnp.concatenate([h[:_N] for h in hs], axis=-1)


# speedup vs baseline: 1.2223x; 1.0723x over previous
import functools

import jax
import jax.numpy as jnp
from jax import lax
from jax.experimental import pallas as pl
from jax.experimental.pallas import tpu as pltpu
from jax.experimental.pallas import tpu_sc as plsc

_N = 10000
_E = 320000
_D = 128
_NC = 2
_NS = 16
_NW = _NC * _NS
_EPT = 10240
_EP = _NW * _EPT
_HALF = 5120
_SINK = 16
_AR = _HALF + _SINK
_SEG = _EPT + 32
_K = 16
_NB = 4
_NI = 8

_mesh = plsc.VectorSubcoreMesh(core_axis_name="c", subcore_axis_name="s")


@functools.partial(
    pl.kernel,
    mesh=_mesh,
    compiler_params=pltpu.CompilerParams(needs_layout_passes=False),
    out_type=(
        jax.ShapeDtypeStruct((2 * _NW * _SEG,), jnp.int32),
        jax.ShapeDtypeStruct((2 * _NW * _SEG,), jnp.int32),
        jax.ShapeDtypeStruct((32 * _NW,), jnp.int32),
    ),
    scratch_types=[
        pltpu.VMEM((_EPT,), jnp.int32),
        pltpu.VMEM((_EPT,), jnp.int32),
        [pltpu.VMEM((_SEG,), jnp.int32) for _ in range(2)],
        [pltpu.VMEM((_SEG,), jnp.int32) for _ in range(2)],
        pltpu.VMEM((32,), jnp.int32),
    ],
)
def _sc_partition(src_hbm, dst_hbm, sp_hbm, dp_hbm, cnt_hbm,
                  sbuf, dbuf, sc_, dc_, cbuf):
    cid = lax.axis_index("c")
    sid = lax.axis_index("s")
    wid = sid * _NC + cid

    pltpu.sync_copy(src_hbm.at[pl.ds(wid * _EPT, _EPT)], sbuf)
    pltpu.sync_copy(dst_hbm.at[pl.ds(wid * _EPT, _EPT)], dbuf)

    iota = lax.iota(jnp.int32, 16)

    def _body(j, carry):
        p0, p1 = carry
        s = sbuf[pl.ds(j * 16, 16)]
        d = dbuf[pl.ds(j * 16, 16)]
        m0 = d < _HALF
        cv = plsc.cumsum(m0.astype(jnp.int32))
        idx0 = (p0 - 1) + cv
        idx1 = (p1 + iota) - cv
        m1 = jnp.logical_not(m0)
        plsc.store_scatter(sc_[0], [idx0], s, mask=m0)
        plsc.store_scatter(dc_[0], [idx0], d, mask=m0)
        plsc.store_scatter(sc_[1], [idx1], s, mask=m1)
        plsc.store_scatter(dc_[1], [idx1], d - _HALF, mask=m1)
        n0 = jnp.max(cv)
        return p0 + n0, p1 + (16 - n0)

    p0, p1 = lax.fori_loop(0, _EPT // 16, _body, (0, 0))

    sinkd = _HALF + iota
    zsrc = jnp.zeros((16,), jnp.int32)
    for h, p in ((0, p0), (1, p1)):
        for o in (0, 16):
            plsc.store_scatter(dc_[h], [p + o + iota], sinkd)
            plsc.store_scatter(sc_[h], [p + o + iota], zsrc)
    p0c = ((p0 + 31) // 32) * 32
    p1c = ((p1 + 31) // 32) * 32
    cbuf[pl.ds(0, 16)] = jnp.full((16,), 0, jnp.int32) + p0c
    cbuf[pl.ds(16, 16)] = jnp.full((16,), 0, jnp.int32) + p1c

    pltpu.sync_copy(cbuf, cnt_hbm.at[pl.ds(wid * 32, 32)])
    for h in (0, 1):
        base = (wid * 2 + h) * _SEG
        pltpu.sync_copy(sc_[h], sp_hbm.at[pl.ds(base, _SEG)])
        pltpu.sync_copy(dc_[h], dp_hbm.at[pl.ds(base, _SEG)])


@functools.partial(
    pl.kernel,
    mesh=_mesh,
    compiler_params=pltpu.CompilerParams(needs_layout_passes=False),
    out_type=jax.ShapeDtypeStruct((_NC, _HALF, _D), jnp.float32),
    scratch_types=[
        pltpu.VMEM((_NI * _K,), jnp.int32),
        [pltpu.VMEM((_K,), jnp.int32) for _ in range(_NI)],
        pltpu.VMEM((_NB, _K, _D), jnp.float32),
        pltpu.VMEM_SHARED((_N, _D), jnp.float32),
        pltpu.VMEM_SHARED((_AR, _D), jnp.float32),
        [pltpu.SemaphoreType.DMA for _ in range(_NI)],
        [pltpu.SemaphoreType.DMA for _ in range(_NB)],
        [pltpu.SemaphoreType.DMA for _ in range(_NB)],
    ],
)
def _sc_aggregate(h_hbm, sp_hbm, dp_hbm, cnt_hbm, out_hbm,
                  srcm, dst_i, rows, h_sh, agg_sh, isems, rsems, ssems):
    cid = lax.axis_index("c")
    sid = lax.axis_index("s")

    cb0 = (4 * sid + cid) * _SEG
    cb1 = (4 * sid + 2 + cid) * _SEG

    pltpu.sync_copy(cnt_hbm.at[pl.ds((2 * sid) * 32 + cid * 16, 16)],
                    srcm.at[pl.ds(0, 16)])
    pltpu.sync_copy(cnt_hbm.at[pl.ds((2 * sid + 1) * 32 + cid * 16, 16)],
                    srcm.at[pl.ds(16, 16)])
    n0 = jnp.max(srcm[pl.ds(0, 16)])
    n1 = jnp.max(srcm[pl.ds(16, 16)])
    c0 = n0 // _K
    t = c0 + n1 // _K
    g_hi = (t + 2 + _NI - 1) // _NI

    def _off(j):
        return jnp.where(j < c0, cb0 + j * _K, cb1 + (j - c0) * _K)

    def _start_idx(j, slot):
        @pl.when(j < t)
        def _():
            o = _off(j)
            pltpu.async_copy(sp_hbm.at[pl.ds(o, _K)],
                             srcm.at[pl.ds(slot * _K, _K)], isems[slot])
            pltpu.async_copy(dp_hbm.at[pl.ds(o, _K)], dst_i[slot], isems[slot])

    def _wait_idx(j, slot):
        @pl.when(j < t)
        def _():
            pltpu.make_async_copy(sp_hbm.at[pl.ds(0, _K)],
                                  srcm.at[pl.ds(slot * _K, _K)],
                                  isems[slot]).wait()
            pltpu.make_async_copy(dp_hbm.at[pl.ds(0, _K)], dst_i[slot],
                                  isems[slot]).wait()

    def _start_gather(j, slot, rslot):
        @pl.when(j < t)
        def _():
            pltpu.async_copy(h_sh.at[srcm.at[pl.ds(slot * _K, _K)]],
                             rows.at[rslot], rsems[rslot])

    def _wait_gather(j, rslot):
        @pl.when(j < t)
        def _():
            pltpu.make_async_copy(h_sh.at[srcm.at[pl.ds(0, _K)]], rows.at[rslot],
                                  rsems[rslot]).wait()

    def _start_scatter(j, slot, rslot):
        @pl.when(j < t)
        def _():
            pltpu.async_copy(rows.at[rslot], agg_sh.at[dst_i[slot]],
                             ssems[rslot], add=True)

    def _wait_scatter(j, rslot):
        @pl.when(jnp.logical_and(j >= 0, j < t))
        def _():
            pltpu.make_async_copy(rows.at[rslot], agg_sh.at[dst_i[0]],
                                  ssems[rslot]).wait()

    @pl.when(sid < 15)
    def _():
        pltpu.sync_copy(h_hbm.at[pl.ds(sid * 632, 632)],
                        h_sh.at[pl.ds(sid * 632, 632)])

    @pl.when(sid == 15)
    def _():
        pltpu.sync_copy(h_hbm.at[pl.ds(9480, 520)], h_sh.at[pl.ds(9480, 520)])

    def _zbody(i, carry):
        r = i // (_D // 16)
        c = (i % (_D // 16)) * 16
        rows[0, r, pl.ds(c, 16)] = jnp.zeros((16,), jnp.float32)
        return carry

    lax.fori_loop(0, _K * (_D // 16), _zbody, 0)
    zb = sid * (_AR // _NS)
    for q in range(_AR // _NS // _K):
        pltpu.sync_copy(rows.at[0], agg_sh.at[pl.ds(zb + q * _K, _K)])
    pltpu.sync_copy(rows.at[0, pl.ds(0, _AR // _NS % _K)],
                    agg_sh.at[pl.ds(zb + (_AR // _NS // _K) * _K,
                                    _AR // _NS % _K)])
    plsc.subcore_barrier()

    for j in range(6):
        _start_idx(j, j)
    for j in range(2):
        _wait_idx(j, j)
        _start_gather(j, j, j)

    def _body(g, carry):
        for b in range(_NI):
            j = g * _NI + b
            a = b % _NB

            _wait_gather(j, a)
            _start_scatter(j, b, a)
            _wait_scatter(j - 2, (b + 2) % _NB)
            _start_idx(j + 6, (b + 6) % _NI)
            _wait_idx(j + 2, (b + 2) % _NI)
            _start_gather(j + 2, (b + 2) % _NI, (b + 2) % _NB)
        return carry

    lax.fori_loop(0, g_hi, _body, 0)
    plsc.subcore_barrier()

    pltpu.sync_copy(agg_sh.at[pl.ds(sid * (_HALF // _NS), _HALF // _NS)],
                    out_hbm.at[cid, pl.ds(sid * (_HALF // _NS), _HALF // _NS)])


_NP = _NC * _HALF
_RB = 1024


def _mlp_body(h_ref, p_ref, w1_ref, b1_ref, w2_ref, b2_ref, o_ref):
    z = h_ref[...] + p_ref[0]
    z = jnp.dot(z, w1_ref[...], preferred_element_type=jnp.float32) + b1_ref[...]
    z = jnp.maximum(z, 0.0)
    o_ref[...] = (jnp.dot(z, w2_ref[...], preferred_element_type=jnp.float32)
                  + b2_ref[...])


def _mlp(h, p, w1, b1, w2, b2):
    return pl.pallas_call(
        _mlp_body,
        grid=(_NP // _RB,),
        in_specs=[
            pl.BlockSpec((_RB, _D), lambda i: (i, 0)),
            pl.BlockSpec((1, _RB, _D), lambda i: (i // 5, i % 5, 0)),
            pl.BlockSpec((_D, _D), lambda i: (0, 0)),
            pl.BlockSpec((1, _D), lambda i: (0, 0)),
            pl.BlockSpec((_D, _D), lambda i: (0, 0)),
            pl.BlockSpec((1, _D), lambda i: (0, 0)),
        ],
        out_specs=pl.BlockSpec((_RB, _D), lambda i: (i, 0)),
        out_shape=jax.ShapeDtypeStruct((_NP, _D), jnp.float32),
    )(h, p, w1, b1.reshape(1, _D), w2, b2.reshape(1, _D))


def kernel(x, edge_index, W1_0, b1_0, W2_0, b2_0, W1_1, b1_1, W2_1, b2_1,
           W1_2, b1_2, W2_2, b2_2):
    pad = _EP - _E
    src = jnp.concatenate(
        [edge_index[0].astype(jnp.int32), jnp.zeros((pad,), jnp.int32)])
    dst = jnp.concatenate(
        [edge_index[1].astype(jnp.int32), jnp.full((pad,), _N, jnp.int32)])
    spf, dpf, cnt = _sc_partition(src, dst)
    params = [(W1_0, b1_0, W2_0, b2_0), (W1_1, b1_1, W2_1, b2_1),
              (W1_2, b1_2, W2_2, b2_2)]
    x_p = jnp.concatenate([x, jnp.zeros((_NP - _N, _D), jnp.float32)])
    hs = [x_p]
    for (w1, b1, w2, b2) in params:
        p = _sc_aggregate(hs[-1], spf, dpf, cnt)
        hs.append(_mlp(hs[-1], p, w1, b1, w2, b2))
    return jnp.concatenate([h[:_N] for h in hs], axis=-1)
